# baseline (speedup 1.0000x reference)
import functools

import jax
import jax.numpy as jnp
from jax.experimental import pallas as pl
from jax.experimental.pallas import tpu as pltpu
from jax.experimental.pallas import tpu_sc as plsc

_N = 8192
_K = 8192
_D = 256
_TN = 512
_TK = 4096
_TKB = 4096
_CC = 0.25

_SC_NC = 2
_SC_NS = 16
_SC_NW = _SC_NC * _SC_NS
_GB = _N // _SC_NW


def _fused_body(x_ref, w_ref, iota_ref,
                idx_ref, loss_ref, enc_ref, perp_ref,
                rmin_ref, ridx2_ref, cnt_ref, acc_ref):
    n = pl.program_id(0)
    k = pl.program_id(1)
    nn = pl.num_programs(0)
    nk = pl.num_programs(1)
    cur = jax.lax.rem(n, 2)
    prev = 1 - cur

    x = x_ref[...]
    w = w_ref[...]
    xsq = jnp.sum(x * x, axis=1, keepdims=True)
    wsq = jnp.sum(w * w, axis=1)[None, :]
    mm = jax.lax.dot_general(x, w, (((1,), (1,)), ((), ())),
                             preferred_element_type=jnp.float32)
    d = (xsq + wsq) - 2.0 * mm
    bmin = jnp.min(d, axis=1, keepdims=True)
    bidx = jnp.min(jnp.where(d == bmin, iota_ref[...], jnp.float32(3e38)),
                   axis=1, keepdims=True)

    @pl.when(k == 0)
    def _():
        rmin_ref[...] = bmin
        ridx2_ref[cur] = bidx

    @pl.when(k > 0)
    def _():
        upd = bmin < rmin_ref[...]
        ridx2_ref[cur] = jnp.where(upd, bidx, ridx2_ref[cur])
        rmin_ref[...] = jnp.where(upd, bmin, rmin_ref[...])

    @pl.when(k == nk - 1)
    def _():
        idx_ref[...] = ridx2_ref[cur].astype(jnp.int32)

        @pl.when(n < nn - 1)
        def _():
            s = jnp.sum(rmin_ref[...])

            @pl.when(n == 0)
            def _():
                acc_ref[0] = s

            @pl.when(n > 0)
            def _():
                acc_ref[0] += s

    oh = jnp.where(ridx2_ref[prev] == iota_ref[...],
                   jnp.float32(1.0), jnp.float32(0.0))
    enc_ref[...] = oh
    csum = jnp.sum(oh, axis=0, keepdims=True)

    @pl.when(n == 1)
    def _():
        cnt_ref[:, pl.ds(k * _TKB, _TKB)] = csum

    @pl.when(n > 1)
    def _():
        cnt_ref[:, pl.ds(k * _TKB, _TKB)] += csum

    @pl.when(jnp.logical_and(n == nn - 1, k == nk - 1))
    def _():
        loss = _CC * acc_ref[0] / (_N * _D)
        loss_ref[...] = jnp.broadcast_to(loss, (1, 1))
        p = cnt_ref[...] * jnp.float32(1.0 / _N)
        ent = jnp.sum(p * jnp.log(p + 1e-10))
        perp_ref[...] = jnp.broadcast_to(jnp.exp(-ent), (1, 1))


def _fused(flat, w, iota_f32):
    nb = _N // _TN
    return pl.pallas_call(
        _fused_body,
        grid=(nb + 1, _K // _TK),
        in_specs=[pl.BlockSpec((_TN, _D),
                               lambda n, k: (jnp.minimum(n, nb - 1), 0)),
                  pl.BlockSpec((_TK, _D), lambda n, k: (k, 0)),
                  pl.BlockSpec((1, _TK), lambda n, k: (0, k))],
        out_specs=[pl.BlockSpec((_TN, 1),
                                lambda n, k: (jnp.minimum(n, nb - 1), 0)),
                   pl.BlockSpec((1, 1), lambda n, k: (0, 0)),
                   pl.BlockSpec((_TN, _TKB),
                                lambda n, k: (jnp.maximum(n - 1, 0),
                                              jnp.where(n == 0, 0, k))),
                   pl.BlockSpec((1, 1), lambda n, k: (0, 0))],
        out_shape=[jax.ShapeDtypeStruct((_N, 1), jnp.int32),
                   jax.ShapeDtypeStruct((1, 1), jnp.float32),
                   jax.ShapeDtypeStruct((_N, _K), jnp.float32),
                   jax.ShapeDtypeStruct((1, 1), jnp.float32)],
        scratch_shapes=[pltpu.VMEM((_TN, 1), jnp.float32),
                        pltpu.VMEM((2, _TN, 1), jnp.float32),
                        pltpu.VMEM((1, _K), jnp.float32),
                        pltpu.SMEM((1,), jnp.float32)],
    )(flat, w, iota_f32)


def _sc_gather(table, idx_flat):
    mesh = plsc.VectorSubcoreMesh(core_axis_name="c", subcore_axis_name="s")

    @functools.partial(
        pl.kernel,
        mesh=mesh,
        out_type=jax.ShapeDtypeStruct((_N, _D), jnp.float32),
        scratch_types=[pltpu.VMEM((_GB,), jnp.int32),
                       pltpu.VMEM((_GB, _D), jnp.float32),
                       pltpu.SemaphoreType.DMA],
    )
    def g(table_hbm, idx_hbm, out_hbm, idx_v, rows_v, sem):
        wid = jax.lax.axis_index("s") * _SC_NC + jax.lax.axis_index("c")
        base = wid * _GB
        pltpu.sync_copy(idx_hbm.at[pl.ds(base, _GB)], idx_v)
        pltpu.async_copy(table_hbm.at[idx_v], rows_v, sem).wait()
        pltpu.sync_copy(rows_v, out_hbm.at[pl.ds(base, _GB)])

    return g(table, idx_flat)


def kernel(inputs, embedding_weight):
    flat = jnp.transpose(inputs, (0, 2, 3, 1)).reshape(_N, _D)
    iota_f32 = jax.lax.iota(jnp.float32, _K).reshape(1, _K)
    idx, loss11, enc, perp11 = _fused(flat, embedding_weight, iota_f32)
    idx_flat = idx.reshape(_N)
    q = _sc_gather(embedding_weight, idx_flat)
    quantized_out = jnp.transpose(q.reshape(8, 32, 32, _D), (0, 3, 1, 2))
    indices = idx_flat.reshape(8, 32, 32)
    return (loss11[0, 0], quantized_out, perp11[0, 0], enc, indices)


# speedup vs baseline: 1.0412x; 1.0412x over previous
import functools

import jax
import jax.numpy as jnp
from jax.experimental import pallas as pl
from jax.experimental.pallas import tpu as pltpu
from jax.experimental.pallas import tpu_sc as plsc

_N = 8192
_K = 8192
_D = 256
_TN = 1024
_TK = 2048
_TKB = 2048
_CC = 0.25

_SC_NC = 2
_SC_NS = 16
_SC_NW = _SC_NC * _SC_NS
_GB = _N // _SC_NW


def _fused_body(x_ref, w_ref, iota_ref,
                idx_ref, loss_ref, enc_ref, perp_ref,
                rmin_ref, ridx2_ref, cnt_ref, acc_ref):
    n = pl.program_id(0)
    k = pl.program_id(1)
    nn = pl.num_programs(0)
    nk = pl.num_programs(1)
    cur = jax.lax.rem(n, 2)
    prev = 1 - cur

    @pl.when(n < nn - 1)
    def _():
        x = x_ref[...]
        w = w_ref[...]
        xsq = jnp.sum(x * x, axis=1, keepdims=True)
        wsq = jnp.sum(w * w, axis=1)[None, :]
        mm = jax.lax.dot_general(x, w, (((1,), (1,)), ((), ())),
                                 preferred_element_type=jnp.float32)
        d = (xsq + wsq) - 2.0 * mm
        bmin = jnp.min(d, axis=1, keepdims=True)
        bidx = jnp.min(jnp.where(d == bmin, iota_ref[...], jnp.float32(3e38)),
                       axis=1, keepdims=True)

        @pl.when(k == 0)
        def _():
            rmin_ref[...] = bmin
            ridx2_ref[cur] = bidx

        @pl.when(k > 0)
        def _():
            upd = bmin < rmin_ref[...]
            ridx2_ref[cur] = jnp.where(upd, bidx, ridx2_ref[cur])
            rmin_ref[...] = jnp.where(upd, bmin, rmin_ref[...])

        @pl.when(k == nk - 1)
        def _():
            idx_ref[...] = ridx2_ref[cur].astype(jnp.int32)
            s = jnp.sum(rmin_ref[...])

            @pl.when(n == 0)
            def _():
                acc_ref[0] = s

            @pl.when(n > 0)
            def _():
                acc_ref[0] += s

    oh = jnp.where(ridx2_ref[prev] == iota_ref[...],
                   jnp.float32(1.0), jnp.float32(0.0))
    enc_ref[...] = oh
    csum = jnp.sum(oh, axis=0, keepdims=True)

    @pl.when(n == 1)
    def _():
        cnt_ref[:, pl.ds(k * _TKB, _TKB)] = csum

    @pl.when(n > 1)
    def _():
        cnt_ref[:, pl.ds(k * _TKB, _TKB)] += csum

    @pl.when(jnp.logical_and(n == nn - 1, k == nk - 1))
    def _():
        loss = _CC * acc_ref[0] / (_N * _D)
        loss_ref[...] = jnp.broadcast_to(loss, (1, 1))
        p = cnt_ref[...] * jnp.float32(1.0 / _N)
        ent = jnp.sum(p * jnp.log(p + 1e-10))
        perp_ref[...] = jnp.broadcast_to(jnp.exp(-ent), (1, 1))


def _fused(flat, w, iota_f32):
    nb = _N // _TN
    return pl.pallas_call(
        _fused_body,
        grid=(nb + 1, _K // _TK),
        in_specs=[pl.BlockSpec((_TN, _D),
                               lambda n, k: (jnp.minimum(n, nb - 1), 0)),
                  pl.BlockSpec((_TK, _D), lambda n, k: (k, 0)),
                  pl.BlockSpec((1, _TK), lambda n, k: (0, k))],
        out_specs=[pl.BlockSpec((_TN, 1),
                                lambda n, k: (jnp.minimum(n, nb - 1), 0)),
                   pl.BlockSpec((1, 1), lambda n, k: (0, 0)),
                   pl.BlockSpec((_TN, _TKB),
                                lambda n, k: (jnp.maximum(n - 1, 0),
                                              jnp.where(n == 0, 0, k))),
                   pl.BlockSpec((1, 1), lambda n, k: (0, 0))],
        out_shape=[jax.ShapeDtypeStruct((_N, 1), jnp.int32),
                   jax.ShapeDtypeStruct((1, 1), jnp.float32),
                   jax.ShapeDtypeStruct((_N, _K), jnp.float32),
                   jax.ShapeDtypeStruct((1, 1), jnp.float32)],
        scratch_shapes=[pltpu.VMEM((_TN, 1), jnp.float32),
                        pltpu.VMEM((2, _TN, 1), jnp.float32),
                        pltpu.VMEM((1, _K), jnp.float32),
                        pltpu.SMEM((1,), jnp.float32)],
    )(flat, w, iota_f32)


def _sc_gather(table, idx_flat):
    mesh = plsc.VectorSubcoreMesh(core_axis_name="c", subcore_axis_name="s")

    @functools.partial(
        pl.kernel,
        mesh=mesh,
        out_type=jax.ShapeDtypeStruct((_N, _D), jnp.float32),
        scratch_types=[pltpu.VMEM((_GB,), jnp.int32),
                       pltpu.VMEM((_GB, _D), jnp.float32),
                       pltpu.SemaphoreType.DMA],
    )
    def g(table_hbm, idx_hbm, out_hbm, idx_v, rows_v, sem):
        wid = jax.lax.axis_index("s") * _SC_NC + jax.lax.axis_index("c")
        base = wid * _GB
        pltpu.sync_copy(idx_hbm.at[pl.ds(base, _GB)], idx_v)
        pltpu.async_copy(table_hbm.at[idx_v], rows_v, sem).wait()
        pltpu.sync_copy(rows_v, out_hbm.at[pl.ds(base, _GB)])

    return g(table, idx_flat)


def kernel(inputs, embedding_weight):
    flat = jnp.transpose(inputs, (0, 2, 3, 1)).reshape(_N, _D)
    iota_f32 = jax.lax.iota(jnp.float32, _K).reshape(1, _K)
    idx, loss11, enc, perp11 = _fused(flat, embedding_weight, iota_f32)
    idx_flat = idx.reshape(_N)
    q = _sc_gather(embedding_weight, idx_flat)
    quantized_out = jnp.transpose(q.reshape(8, 32, 32, _D), (0, 3, 1, 2))
    indices = idx_flat.reshape(8, 32, 32)
    return (loss11[0, 0], quantized_out, perp11[0, 0], enc, indices)
